# Initial kernel scaffold; baseline (speedup 1.0000x reference)
import functools

import jax
import jax.numpy as jnp
from jax import lax
from jax.experimental import pallas as pl
from jax.experimental.pallas import tpu as pltpu
from jax.experimental.pallas import tpu_sc as plsc

VOCAB = 100000
D = 300
H = 768
EPS = 1e-12
B, L = 4096, 50
N = B * L

NC, NS = 2, 16
NW = NC * NS
ROWS_PER_W = N // NW
C = 128
CHUNKS = ROWS_PER_W // C

_sc_mesh = plsc.VectorSubcoreMesh(core_axis_name="c", subcore_axis_name="s")


@functools.partial(
    pl.kernel,
    out_type=jax.ShapeDtypeStruct((N, D), jnp.float32),
    mesh=_sc_mesh,
    scratch_types=[
        pltpu.VMEM((C,), jnp.int32),
        pltpu.VMEM((C, D), jnp.float32),
        pltpu.SemaphoreType.DMA,
    ],
)
def _sc_gather(tbl_hbm, idx_hbm, out_hbm, idx_v, rows_v, gsem):
    wid = lax.axis_index("s") * NC + lax.axis_index("c")
    base = wid * ROWS_PER_W

    def step(i, carry):
        off = base + i * C
        pltpu.sync_copy(idx_hbm.at[pl.ds(off, C)], idx_v)
        pltpu.async_copy(tbl_hbm.at[idx_v], rows_v, gsem).wait()
        pltpu.sync_copy(rows_v, out_hbm.at[pl.ds(off, C)])
        return carry

    lax.fori_loop(0, CHUNKS, step, 0)


T = 1024


def _tc_body(x_ref, g1_ref, b1_ref, w_ref, b_ref, g2_ref, b2_ref, o_ref):
    x = x_ref[...]
    mu = jnp.mean(x, axis=-1, keepdims=True)
    var = jnp.mean((x - mu) ** 2, axis=-1, keepdims=True)
    xh = (x - mu) * lax.rsqrt(var + EPS)
    xh = xh * g1_ref[...] + b1_ref[...]
    y = jnp.dot(xh, w_ref[...], preferred_element_type=jnp.float32) + b_ref[...]
    y = jnp.maximum(y, 0.0)
    mu2 = jnp.mean(y, axis=-1, keepdims=True)
    var2 = jnp.mean((y - mu2) ** 2, axis=-1, keepdims=True)
    o_ref[...] = (y - mu2) * lax.rsqrt(var2 + EPS) * g2_ref[...] + b2_ref[...]


def _tc_dense(x, g1, b1, w, b, g2, b2):
    grid = (N // T,)
    return pl.pallas_call(
        _tc_body,
        grid=grid,
        in_specs=[
            pl.BlockSpec((T, D), lambda i: (i, 0)),
            pl.BlockSpec((1, D), lambda i: (0, 0)),
            pl.BlockSpec((1, D), lambda i: (0, 0)),
            pl.BlockSpec((D, H), lambda i: (0, 0)),
            pl.BlockSpec((1, H), lambda i: (0, 0)),
            pl.BlockSpec((1, H), lambda i: (0, 0)),
            pl.BlockSpec((1, H), lambda i: (0, 0)),
        ],
        out_specs=pl.BlockSpec((T, H), lambda i: (i, 0)),
        out_shape=jax.ShapeDtypeStruct((N, H), jnp.float32),
    )(x, g1.reshape(1, D), b1.reshape(1, D), w, b.reshape(1, H),
      g2.reshape(1, H), b2.reshape(1, H))


def kernel(input_ids, emb_table, ln1_gamma, ln1_beta, W, b, ln2_gamma, ln2_beta):
    ids = input_ids.reshape(-1).astype(jnp.int32)
    gathered = _sc_gather(emb_table, ids)
    out = _tc_dense(gathered, ln1_gamma, ln1_beta, W, b, ln2_gamma, ln2_beta)
    return out.reshape(B, L, H)


# speedup vs baseline: 1.1509x; 1.1509x over previous
import functools

import jax
import jax.numpy as jnp
from jax import lax
from jax.experimental import pallas as pl
from jax.experimental.pallas import tpu as pltpu
from jax.experimental.pallas import tpu_sc as plsc

VOCAB = 100000
D = 300
DP = 384
H = 768
EPS = 1e-12
B, L = 4096, 50
N = B * L

NC, NS = 2, 16
NW = NC * NS
ROWS_PER_W = N // NW
C = 128
CHUNKS = ROWS_PER_W // C

_sc_mesh = plsc.VectorSubcoreMesh(core_axis_name="c", subcore_axis_name="s")


@functools.partial(
    pl.kernel,
    out_type=jax.ShapeDtypeStruct((N, DP), jnp.float32),
    mesh=_sc_mesh,
    scratch_types=[
        pltpu.VMEM((C,), jnp.int32),
        pltpu.VMEM((C, DP), jnp.float32),
        pltpu.SemaphoreType.DMA,
    ],
)
def _sc_gather(tbl_hbm, idx_hbm, out_hbm, idx_v, rows_v, gsem):
    wid = lax.axis_index("s") * NC + lax.axis_index("c")
    base = wid * ROWS_PER_W

    def step(i, carry):
        off = base + i * C
        pltpu.sync_copy(idx_hbm.at[pl.ds(off, C)], idx_v)
        pltpu.async_copy(tbl_hbm.at[idx_v], rows_v, gsem).wait()
        pltpu.sync_copy(rows_v, out_hbm.at[pl.ds(off, C)])
        return carry

    lax.fori_loop(0, CHUNKS, step, 0)


T = 1024


def _tc_body(x_ref, g1_ref, b1_ref, w_ref, b_ref, g2_ref, b2_ref, o_ref):
    x = x_ref[...]
    s1 = jnp.sum(x, axis=-1, keepdims=True)
    s2 = jnp.sum(x * x, axis=-1, keepdims=True)
    mu = s1 * (1.0 / D)
    var = s2 * (1.0 / D) - mu * mu
    xh = (x - mu) * lax.rsqrt(var + EPS)
    xh = xh * g1_ref[...] + b1_ref[...]
    y = jnp.dot(xh, w_ref[...], preferred_element_type=jnp.float32) + b_ref[...]
    y = jnp.maximum(y, 0.0)
    mu2 = jnp.mean(y, axis=-1, keepdims=True)
    var2 = jnp.mean((y - mu2) ** 2, axis=-1, keepdims=True)
    o_ref[...] = (y - mu2) * lax.rsqrt(var2 + EPS) * g2_ref[...] + b2_ref[...]


def _tc_dense(x, g1, b1, w, b, g2, b2):
    grid = (N // T,)
    return pl.pallas_call(
        _tc_body,
        grid=grid,
        in_specs=[
            pl.BlockSpec((T, DP), lambda i: (i, 0)),
            pl.BlockSpec((1, DP), lambda i: (0, 0)),
            pl.BlockSpec((1, DP), lambda i: (0, 0)),
            pl.BlockSpec((DP, H), lambda i: (0, 0)),
            pl.BlockSpec((1, H), lambda i: (0, 0)),
            pl.BlockSpec((1, H), lambda i: (0, 0)),
            pl.BlockSpec((1, H), lambda i: (0, 0)),
        ],
        out_specs=pl.BlockSpec((T, H), lambda i: (i, 0)),
        out_shape=jax.ShapeDtypeStruct((N, H), jnp.float32),
    )(x, g1.reshape(1, DP), b1.reshape(1, DP), w, b.reshape(1, H),
      g2.reshape(1, H), b2.reshape(1, H))


def kernel(input_ids, emb_table, ln1_gamma, ln1_beta, W, b, ln2_gamma, ln2_beta):
    ids = input_ids.reshape(-1).astype(jnp.int32)
    pad = DP - D
    tbl = jnp.pad(emb_table, ((0, 0), (0, pad)))
    g1 = jnp.pad(ln1_gamma, (0, pad))
    b1 = jnp.pad(ln1_beta, (0, pad))
    w = jnp.pad(W, ((0, pad), (0, 0)))
    gathered = _sc_gather(tbl, ids)
    out = _tc_dense(gathered, g1, b1, w, b, ln2_gamma, ln2_beta)
    return out.reshape(B, L, H)


# speedup vs baseline: 3.0449x; 2.6457x over previous
import functools

import jax
import jax.numpy as jnp
from jax import lax
from jax.experimental import pallas as pl
from jax.experimental.pallas import tpu as pltpu
from jax.experimental.pallas import tpu_sc as plsc

VOCAB = 100000
D = 300
DP = 384
H = 768
EPS = 1e-12
B, L = 4096, 50
N = B * L

NC, NS = 2, 16
NW = NC * NS
ROWS_PER_W = N // NW
C = 128
CHUNKS = ROWS_PER_W // C

_sc_mesh = plsc.VectorSubcoreMesh(core_axis_name="c", subcore_axis_name="s")


@functools.partial(
    pl.kernel,
    out_type=jax.ShapeDtypeStruct((N, DP), jnp.float32),
    mesh=_sc_mesh,
    scratch_types=[
        pltpu.VMEM((C,), jnp.int32),
        pltpu.VMEM((C, DP), jnp.float32),
        pltpu.SemaphoreType.DMA,
    ],
)
def _sc_gather(tbl_hbm, idx_hbm, out_hbm, idx_v, rows_v, gsem):
    wid = lax.axis_index("s") * NC + lax.axis_index("c")
    base = wid * ROWS_PER_W

    def step(i, carry):
        off = base + i * C
        pltpu.sync_copy(idx_hbm.at[pl.ds(off, C)], idx_v)
        pltpu.async_copy(tbl_hbm.at[idx_v], rows_v, gsem).wait()
        pltpu.sync_copy(rows_v, out_hbm.at[pl.ds(off, C)])
        return carry

    lax.fori_loop(0, CHUNKS, step, 0)


VB = 1024


def _tp_body(xt_ref, i_ref, o_ref):
    o_ref[...] = lax.dot_general(
        xt_ref[...], i_ref[...],
        dimension_numbers=(((0,), (0,)), ((), ())),
        preferred_element_type=jnp.float32,
    )


def _tc_transpose_pad(tbl_t, eye):
    return pl.pallas_call(
        _tp_body,
        grid=((VOCAB + VB - 1) // VB,),
        in_specs=[
            pl.BlockSpec((D, VB), lambda i: (0, i)),
            pl.BlockSpec((D, DP), lambda i: (0, 0)),
        ],
        out_specs=pl.BlockSpec((VB, DP), lambda i: (i, 0)),
        out_shape=jax.ShapeDtypeStruct((VOCAB, DP), jnp.float32),
    )(tbl_t, eye)


T = 1024


def _tc_body(x_ref, g1_ref, b1_ref, w_ref, b_ref, g2_ref, b2_ref, o_ref):
    x = x_ref[...]
    s1 = jnp.sum(x, axis=-1, keepdims=True)
    s2 = jnp.sum(x * x, axis=-1, keepdims=True)
    mu = s1 * (1.0 / D)
    var = s2 * (1.0 / D) - mu * mu
    xh = (x - mu) * lax.rsqrt(var + EPS)
    xh = xh * g1_ref[...] + b1_ref[...]
    y = jnp.dot(xh, w_ref[...], preferred_element_type=jnp.float32) + b_ref[...]
    y = jnp.maximum(y, 0.0)
    mu2 = jnp.mean(y, axis=-1, keepdims=True)
    var2 = jnp.mean((y - mu2) ** 2, axis=-1, keepdims=True)
    o_ref[...] = (y - mu2) * lax.rsqrt(var2 + EPS) * g2_ref[...] + b2_ref[...]


def _tc_dense(x, g1, b1, w, b, g2, b2):
    grid = (N // T,)
    return pl.pallas_call(
        _tc_body,
        grid=grid,
        in_specs=[
            pl.BlockSpec((T, DP), lambda i: (i, 0)),
            pl.BlockSpec((1, DP), lambda i: (0, 0)),
            pl.BlockSpec((1, DP), lambda i: (0, 0)),
            pl.BlockSpec((DP, H), lambda i: (0, 0)),
            pl.BlockSpec((1, H), lambda i: (0, 0)),
            pl.BlockSpec((1, H), lambda i: (0, 0)),
            pl.BlockSpec((1, H), lambda i: (0, 0)),
        ],
        out_specs=pl.BlockSpec((T, H), lambda i: (i, 0)),
        out_shape=jax.ShapeDtypeStruct((N, H), jnp.float32),
    )(x, g1.reshape(1, DP), b1.reshape(1, DP), w, b.reshape(1, H),
      g2.reshape(1, H), b2.reshape(1, H))


def kernel(input_ids, emb_table, ln1_gamma, ln1_beta, W, b, ln2_gamma, ln2_beta):
    ids = input_ids.astype(jnp.int32).T.reshape(-1)
    pad = DP - D
    eye = jnp.eye(D, DP, dtype=jnp.float32)
    tbl = _tc_transpose_pad(emb_table.T, eye)
    g1 = jnp.pad(ln1_gamma, (0, pad))
    b1 = jnp.pad(ln1_beta, (0, pad))
    w = jnp.pad(W, ((0, pad), (0, 0)))
    gathered = _sc_gather(tbl, ids)
    out = _tc_dense(gathered, g1, b1, w, b, ln2_gamma, ln2_beta)
    return out.reshape(L, B, H).transpose(1, 0, 2)


# speedup vs baseline: 3.0484x; 1.0011x over previous
import functools

import jax
import jax.numpy as jnp
from jax import lax
from jax.experimental import pallas as pl
from jax.experimental.pallas import tpu as pltpu
from jax.experimental.pallas import tpu_sc as plsc

VOCAB = 100000
D = 300
DP = 384
H = 768
EPS = 1e-12
B, L = 4096, 50
N = B * L

NC, NS = 2, 16
NW = NC * NS
ROWS_PER_W = N // NW
C = 128
CHUNKS = ROWS_PER_W // C

_sc_mesh = plsc.VectorSubcoreMesh(core_axis_name="c", subcore_axis_name="s")


@functools.partial(
    pl.kernel,
    out_type=jax.ShapeDtypeStruct((N, DP), jnp.float32),
    mesh=_sc_mesh,
    scratch_types=[
        pltpu.VMEM((C,), jnp.int32),
        pltpu.VMEM((C, DP), jnp.float32),
        pltpu.SemaphoreType.DMA,
    ],
)
def _sc_gather(tbl_hbm, idx_hbm, out_hbm, idx_v, rows_v, gsem):
    wid = lax.axis_index("s") * NC + lax.axis_index("c")
    base = wid * ROWS_PER_W

    def step(i, carry):
        off = base + i * C
        pltpu.sync_copy(idx_hbm.at[pl.ds(off, C)], idx_v)
        pltpu.async_copy(tbl_hbm.at[idx_v], rows_v, gsem).wait()
        pltpu.sync_copy(rows_v, out_hbm.at[pl.ds(off, C)])
        return carry

    lax.fori_loop(0, CHUNKS, step, 0)


VB = 1024


def _tp_body(xt_ref, i_ref, o_ref):
    o_ref[...] = lax.dot_general(
        xt_ref[...], i_ref[...],
        dimension_numbers=(((0,), (0,)), ((), ())),
        preferred_element_type=jnp.float32,
    )


def _tc_transpose_pad(tbl_t, eye):
    return pl.pallas_call(
        _tp_body,
        grid=((VOCAB + VB - 1) // VB,),
        in_specs=[
            pl.BlockSpec((D, VB), lambda i: (0, i)),
            pl.BlockSpec((D, DP), lambda i: (0, 0)),
        ],
        out_specs=pl.BlockSpec((VB, DP), lambda i: (i, 0)),
        out_shape=jax.ShapeDtypeStruct((VOCAB, DP), jnp.float32),
    )(tbl_t, eye)


T = 1024


def _tc_body(x_ref, g1_ref, b1_ref, w_ref, b_ref, g2_ref, b2_ref, o_ref):
    x = x_ref[...]
    s1 = jnp.sum(x, axis=-1, keepdims=True)
    s2 = jnp.sum(x * x, axis=-1, keepdims=True)
    mu = s1 * (1.0 / D)
    var = s2 * (1.0 / D) - mu * mu
    xh = (x - mu) * lax.rsqrt(var + EPS)
    xh = xh * g1_ref[...] + b1_ref[...]
    y = jnp.dot(xh.astype(jnp.bfloat16), w_ref[...],
                preferred_element_type=jnp.float32) + b_ref[...]
    y = jnp.maximum(y, 0.0)
    mu2 = jnp.mean(y, axis=-1, keepdims=True)
    var2 = jnp.mean((y - mu2) ** 2, axis=-1, keepdims=True)
    o_ref[...] = (y - mu2) * lax.rsqrt(var2 + EPS) * g2_ref[...] + b2_ref[...]


def _tc_dense(x, g1, b1, w, b, g2, b2):
    grid = (N // T,)
    return pl.pallas_call(
        _tc_body,
        grid=grid,
        in_specs=[
            pl.BlockSpec((T, DP), lambda i: (i, 0)),
            pl.BlockSpec((1, DP), lambda i: (0, 0)),
            pl.BlockSpec((1, DP), lambda i: (0, 0)),
            pl.BlockSpec((DP, H), lambda i: (0, 0)),
            pl.BlockSpec((1, H), lambda i: (0, 0)),
            pl.BlockSpec((1, H), lambda i: (0, 0)),
            pl.BlockSpec((1, H), lambda i: (0, 0)),
        ],
        out_specs=pl.BlockSpec((T, H), lambda i: (i, 0)),
        out_shape=jax.ShapeDtypeStruct((N, H), jnp.float32),
    )(x, g1.reshape(1, DP), b1.reshape(1, DP), w, b.reshape(1, H),
      g2.reshape(1, H), b2.reshape(1, H))


def kernel(input_ids, emb_table, ln1_gamma, ln1_beta, W, b, ln2_gamma, ln2_beta):
    ids = input_ids.astype(jnp.int32).T.reshape(-1)
    pad = DP - D
    eye = jnp.eye(D, DP, dtype=jnp.float32)
    tbl = _tc_transpose_pad(emb_table.T, eye)
    g1 = jnp.pad(ln1_gamma, (0, pad))
    b1 = jnp.pad(ln1_beta, (0, pad))
    w = jnp.pad(W, ((0, pad), (0, 0))).astype(jnp.bfloat16)
    gathered = _sc_gather(tbl, ids)
    out = _tc_dense(gathered, g1, b1, w, b, ln2_gamma, ln2_beta)
    return out.reshape(L, B, H).transpose(1, 0, 2)


# speedup vs baseline: 3.4344x; 1.1266x over previous
import functools

import jax
import jax.numpy as jnp
from jax import lax
from jax.experimental import pallas as pl
from jax.experimental.pallas import tpu as pltpu
from jax.experimental.pallas import tpu_sc as plsc

VOCAB = 100000
D = 300
DP = 384
H = 768
EPS = 1e-12
B, L = 4096, 50
N = B * L

NC, NS = 2, 16
NW = NC * NS
S = 5
NCHUNK = N // S
ROWS_PER_W = NCHUNK // NW
C = 128
CHUNKS = ROWS_PER_W // C

assert ROWS_PER_W % C == 0

_sc_mesh = plsc.VectorSubcoreMesh(core_axis_name="c", subcore_axis_name="s")


@functools.partial(
    pl.kernel,
    out_type=jax.ShapeDtypeStruct((NCHUNK, DP), jnp.float32),
    mesh=_sc_mesh,
    scratch_types=[
        pltpu.VMEM((C,), jnp.int32),
        pltpu.VMEM((C, DP), jnp.float32),
        pltpu.SemaphoreType.DMA,
    ],
)
def _sc_gather(tbl_hbm, idx_hbm, out_hbm, idx_v, rows_v, gsem):
    wid = lax.axis_index("s") * NC + lax.axis_index("c")
    base = wid * ROWS_PER_W

    def step(i, carry):
        off = base + i * C
        pltpu.sync_copy(idx_hbm.at[pl.ds(off, C)], idx_v)
        pltpu.async_copy(tbl_hbm.at[idx_v], rows_v, gsem).wait()
        pltpu.sync_copy(rows_v, out_hbm.at[pl.ds(off, C)])
        return carry

    lax.fori_loop(0, CHUNKS, step, 0)


VB = 1024


def _tp_body(xt_ref, i_ref, o_ref):
    o_ref[...] = lax.dot_general(
        xt_ref[...], i_ref[...],
        dimension_numbers=(((0,), (0,)), ((), ())),
        preferred_element_type=jnp.float32,
    )


def _tc_transpose_pad(tbl_t, eye):
    return pl.pallas_call(
        _tp_body,
        grid=((VOCAB + VB - 1) // VB,),
        in_specs=[
            pl.BlockSpec((D, VB), lambda i: (0, i)),
            pl.BlockSpec((D, DP), lambda i: (0, 0)),
        ],
        out_specs=pl.BlockSpec((VB, DP), lambda i: (i, 0)),
        out_shape=jax.ShapeDtypeStruct((VOCAB, DP), jnp.float32),
    )(tbl_t, eye)


T = 1024


def _tc_body(x_ref, g1_ref, b1_ref, w_ref, b_ref, g2_ref, b2_ref, o_ref):
    x = x_ref[...]
    s1 = jnp.sum(x, axis=-1, keepdims=True)
    s2 = jnp.sum(x * x, axis=-1, keepdims=True)
    mu = s1 * (1.0 / D)
    var = s2 * (1.0 / D) - mu * mu
    xh = (x - mu) * lax.rsqrt(var + EPS)
    xh = xh * g1_ref[...] + b1_ref[...]
    y = jnp.dot(xh.astype(jnp.bfloat16), w_ref[...],
                preferred_element_type=jnp.float32) + b_ref[...]
    y = jnp.maximum(y, 0.0)
    mu2 = jnp.mean(y, axis=-1, keepdims=True)
    var2 = jnp.mean((y - mu2) ** 2, axis=-1, keepdims=True)
    o_ref[...] = (y - mu2) * lax.rsqrt(var2 + EPS) * g2_ref[...] + b2_ref[...]


CB = NCHUNK // T


def _acc_body(acc_ref, x_ref, g1_ref, b1_ref, w_ref, b_ref, g2_ref, b2_ref,
              o_ref):
    del acc_ref
    _tc_body(x_ref, g1_ref, b1_ref, w_ref, b_ref, g2_ref, b2_ref, o_ref)


def _tc_dense_chunk(k, acc, x, g1, b1, w, b, g2, b2):
    args = (x, g1.reshape(1, DP), b1.reshape(1, DP), w, b.reshape(1, H),
            g2.reshape(1, H), b2.reshape(1, H))
    in_specs = [
        pl.BlockSpec((T, DP), lambda i: (i, 0)),
        pl.BlockSpec((1, DP), lambda i: (0, 0)),
        pl.BlockSpec((1, DP), lambda i: (0, 0)),
        pl.BlockSpec((DP, H), lambda i: (0, 0)),
        pl.BlockSpec((1, H), lambda i: (0, 0)),
        pl.BlockSpec((1, H), lambda i: (0, 0)),
        pl.BlockSpec((1, H), lambda i: (0, 0)),
    ]
    out_spec = pl.BlockSpec((T, H), lambda i, k=k: (k * CB + i, 0))
    out_shape = jax.ShapeDtypeStruct((N, H), jnp.float32)
    if acc is None:
        return pl.pallas_call(
            _tc_body, grid=(CB,), in_specs=in_specs,
            out_specs=out_spec, out_shape=out_shape,
        )(*args)
    return pl.pallas_call(
        _acc_body, grid=(CB,),
        in_specs=[pl.BlockSpec(memory_space=pl.ANY)] + in_specs,
        out_specs=out_spec, out_shape=out_shape,
        input_output_aliases={0: 0},
    )(acc, *args)


def kernel(input_ids, emb_table, ln1_gamma, ln1_beta, W, b, ln2_gamma, ln2_beta):
    ids = input_ids.astype(jnp.int32).T.reshape(-1)
    pad = DP - D
    eye = jnp.eye(D, DP, dtype=jnp.float32)
    tbl = _tc_transpose_pad(emb_table.T, eye)
    g1 = jnp.pad(ln1_gamma, (0, pad))
    b1 = jnp.pad(ln1_beta, (0, pad))
    w = jnp.pad(W, ((0, pad), (0, 0))).astype(jnp.bfloat16)
    out = None
    for k in range(S):
        ids_k = lax.slice(ids, (k * NCHUNK,), ((k + 1) * NCHUNK,))
        g_k = _sc_gather(tbl, ids_k)
        out = _tc_dense_chunk(k, out, g_k, g1, b1, w, b, ln2_gamma, ln2_beta)
    return out.reshape(L, B, H).transpose(1, 0, 2)


# speedup vs baseline: 3.7524x; 1.0926x over previous
import functools

import jax
import jax.numpy as jnp
from jax import lax
from jax.experimental import pallas as pl
from jax.experimental.pallas import tpu as pltpu
from jax.experimental.pallas import tpu_sc as plsc

VOCAB = 100000
D = 300
DPB = 512
DW = DPB // 2
H = 768
EPS = 1e-12
B, L = 4096, 50
N = B * L

NC, NS = 2, 16
NW = NC * NS
S = 5
NCHUNK = N // S
ROWS_PER_W = NCHUNK // NW
C = 128
CHUNKS = ROWS_PER_W // C

assert ROWS_PER_W % C == 0

_sc_mesh = plsc.VectorSubcoreMesh(core_axis_name="c", subcore_axis_name="s")


@functools.partial(
    pl.kernel,
    out_type=jax.ShapeDtypeStruct((NCHUNK, DW), jnp.float32),
    mesh=_sc_mesh,
    scratch_types=[
        pltpu.VMEM((C,), jnp.int32),
        pltpu.VMEM((C, DW), jnp.float32),
        pltpu.SemaphoreType.DMA,
    ],
)
def _sc_gather(tbl_hbm, idx_hbm, out_hbm, idx_v, rows_v, gsem):
    wid = lax.axis_index("s") * NC + lax.axis_index("c")
    base = wid * ROWS_PER_W

    def step(i, carry):
        off = base + i * C
        pltpu.sync_copy(idx_hbm.at[pl.ds(off, C)], idx_v)
        pltpu.async_copy(tbl_hbm.at[idx_v], rows_v, gsem).wait()
        pltpu.sync_copy(rows_v, out_hbm.at[pl.ds(off, C)])
        return carry

    lax.fori_loop(0, CHUNKS, step, 0)


VB = 1024


def _tp_body(xt_ref, i_ref, o_ref):
    y = lax.dot_general(
        xt_ref[...], i_ref[...],
        dimension_numbers=(((0,), (0,)), ((), ())),
        preferred_element_type=jnp.float32,
    )
    lo = y[:, :DW].astype(jnp.bfloat16).astype(jnp.float32)
    hi = y[:, DW:].astype(jnp.bfloat16).astype(jnp.float32)
    lo_u = lax.bitcast_convert_type(lo, jnp.uint32)
    hi_u = lax.bitcast_convert_type(hi, jnp.uint32)
    word = (hi_u & jnp.uint32(0xFFFF0000)) | (lo_u >> 16)
    o_ref[...] = lax.bitcast_convert_type(word, jnp.float32)


def _tc_transpose_pad(tbl_t, eye):
    return pl.pallas_call(
        _tp_body,
        grid=((VOCAB + VB - 1) // VB,),
        in_specs=[
            pl.BlockSpec((D, VB), lambda i: (0, i)),
            pl.BlockSpec((D, DPB), lambda i: (0, 0)),
        ],
        out_specs=pl.BlockSpec((VB, DW), lambda i: (i, 0)),
        out_shape=jax.ShapeDtypeStruct((VOCAB, DW), jnp.float32),
    )(tbl_t, eye)


T = 1024


def _tc_body(x_ref, w_ref, b_ref, g2_ref, b2_ref, o_ref):
    u = lax.bitcast_convert_type(x_ref[...], jnp.uint32)
    xlo = lax.bitcast_convert_type(u << 16, jnp.float32)
    xhi = lax.bitcast_convert_type(u & jnp.uint32(0xFFFF0000),
                                   jnp.float32)
    s1 = (jnp.sum(xlo, axis=-1, keepdims=True)
          + jnp.sum(xhi, axis=-1, keepdims=True))
    s2 = (jnp.sum(xlo * xlo, axis=-1, keepdims=True)
          + jnp.sum(xhi * xhi, axis=-1, keepdims=True))
    mu = s1 * (1.0 / D)
    var = s2 * (1.0 / D) - mu * mu
    rstd = lax.rsqrt(var + EPS)
    xh_lo = ((xlo - mu) * rstd).astype(jnp.bfloat16)
    xh_hi = ((xhi - mu) * rstd).astype(jnp.bfloat16)
    y = (jnp.dot(xh_lo, w_ref[:DW, :], preferred_element_type=jnp.float32)
         + jnp.dot(xh_hi, w_ref[DW:, :], preferred_element_type=jnp.float32)
         + b_ref[...])
    y = jnp.maximum(y, 0.0)
    mu2 = jnp.mean(y, axis=-1, keepdims=True)
    var2 = jnp.mean((y - mu2) ** 2, axis=-1, keepdims=True)
    o_ref[...] = (y - mu2) * lax.rsqrt(var2 + EPS) * g2_ref[...] + b2_ref[...]


CB = NCHUNK // T


def _acc_body(acc_ref, x_ref, w_ref, b_ref, g2_ref, b2_ref, o_ref):
    del acc_ref
    _tc_body(x_ref, w_ref, b_ref, g2_ref, b2_ref, o_ref)


def _tc_dense_chunk(k, acc, x, w, b, g2, b2):
    args = (x, w, b.reshape(1, H), g2.reshape(1, H), b2.reshape(1, H))
    in_specs = [
        pl.BlockSpec((T, DW), lambda i: (i, 0)),
        pl.BlockSpec((DPB, H), lambda i: (0, 0)),
        pl.BlockSpec((1, H), lambda i: (0, 0)),
        pl.BlockSpec((1, H), lambda i: (0, 0)),
        pl.BlockSpec((1, H), lambda i: (0, 0)),
    ]
    out_spec = pl.BlockSpec((T, H), lambda i, k=k: (k * CB + i, 0))
    out_shape = jax.ShapeDtypeStruct((N, H), jnp.float32)
    if acc is None:
        return pl.pallas_call(
            _tc_body, grid=(CB,), in_specs=in_specs,
            out_specs=out_spec, out_shape=out_shape,
        )(*args)
    return pl.pallas_call(
        _acc_body, grid=(CB,),
        in_specs=[pl.BlockSpec(memory_space=pl.ANY)] + in_specs,
        out_specs=out_spec, out_shape=out_shape,
        input_output_aliases={0: 0},
    )(acc, *args)


def kernel(input_ids, emb_table, ln1_gamma, ln1_beta, W, b, ln2_gamma, ln2_beta):
    ids = input_ids.astype(jnp.int32).T.reshape(-1)
    pad = DPB - D
    eye = jnp.eye(D, DPB, dtype=jnp.float32)
    tbl = _tc_transpose_pad(emb_table.T, eye)
    w = jnp.pad(ln1_gamma[:, None] * W, ((0, pad), (0, 0))).astype(jnp.bfloat16)
    bias = b + ln1_beta @ W
    out = None
    for k in range(S):
        ids_k = lax.slice(ids, (k * NCHUNK,), ((k + 1) * NCHUNK,))
        g_k = _sc_gather(tbl, ids_k)
        out = _tc_dense_chunk(k, out, g_k, w, bias, ln2_gamma, ln2_beta)
    return out.reshape(L, B, H).transpose(1, 0, 2)
